# Initial kernel scaffold; baseline (speedup 1.0000x reference)
import jax
import jax.numpy as jnp
from jax.experimental import pallas as pl


def kernel(x, w_gate, W1, b1, W2, b2, W3, b3):
    raise NotImplementedError("write your pallas kernel here")


# speedup vs baseline: 1.9269x; 1.9269x over previous
import functools

import jax
import jax.numpy as jnp
from jax.experimental import pallas as pl
from jax.experimental.pallas import tpu as pltpu

_E = 8
_K = 2
_N = 2048
_D = 1024
_H = 1024
_TBLK = 512
_NTB = _N // _TBLK


def _gates_for_block(x_blk, wg):
    logits = jnp.dot(x_blk, wg, preferred_element_type=jnp.float32)
    m = jnp.max(logits, axis=1, keepdims=True)
    p = jnp.exp(logits - m)
    p = p / jnp.sum(p, axis=1, keepdims=True)
    lane = jax.lax.broadcasted_iota(jnp.int32, p.shape, 1)
    v0 = jnp.max(p, axis=1, keepdims=True)
    e0 = jnp.min(jnp.where(p == v0, lane, _E), axis=1, keepdims=True)
    mask0 = lane == e0
    p1 = jnp.where(mask0, -1.0, p)
    v1 = jnp.max(p1, axis=1, keepdims=True)
    e1 = jnp.min(jnp.where(p1 == v1, lane, _E), axis=1, keepdims=True)
    mask1 = lane == e1
    denom = v0 + v1 + 1e-6
    return jnp.where(mask0, v0 / denom, 0.0) + jnp.where(mask1, v1 / denom, 0.0)


def _body(x_ref, wg_ref, w1_ref, b1_ref, w2_ref, b2_ref, w3_ref, b3_ref,
          out_ref, acc_ref):
    e = pl.program_id(0)
    t = pl.program_id(1)
    x_blk = x_ref[...]
    gates = _gates_for_block(x_blk, wg_ref[...])
    lane = jax.lax.broadcasted_iota(jnp.int32, gates.shape, 1)
    ge = jnp.sum(jnp.where(lane == e, gates, 0.0), axis=1, keepdims=True)

    scale = jax.lax.shift_left(1, e).astype(jnp.float32)
    h = jnp.tanh(jnp.dot(x_blk * scale, w1_ref[0], preferred_element_type=jnp.float32)
                 + b1_ref[0])
    h = jnp.tanh(jnp.dot(h, w2_ref[0], preferred_element_type=jnp.float32)
                 + b2_ref[0])
    y = jnp.sum(h * w3_ref[0], axis=1, keepdims=True) + b3_ref[0, 0, 0]

    rows = pl.ds(t * _TBLK, _TBLK)

    @pl.when(e == 0)
    def _init():
        acc_ref[rows, :] = ge * y

    @pl.when(e > 0)
    def _acc():
        acc_ref[rows, :] += ge * y

    @pl.when(jnp.logical_and(e == _E - 1, t == _NTB - 1))
    def _emit():
        out_ref[...] = acc_ref[...]


@jax.jit
def kernel(x, w_gate, W1, b1, W2, b2, W3, b3):
    w3r = W3[:, :, 0].reshape(_E, 1, _H)
    b1r = b1.reshape(_E, 1, _H)
    b2r = b2.reshape(_E, 1, _H)
    b3r = b3.reshape(_E, 1, 1)
    grid = (_E, _NTB)
    out = pl.pallas_call(
        _body,
        grid=grid,
        in_specs=[
            pl.BlockSpec((_TBLK, _D), lambda e, t: (t, 0)),
            pl.BlockSpec((_D, _E), lambda e, t: (0, 0)),
            pl.BlockSpec((1, _D, _H), lambda e, t: (e, 0, 0)),
            pl.BlockSpec((1, 1, _H), lambda e, t: (e, 0, 0)),
            pl.BlockSpec((1, _H, _H), lambda e, t: (e, 0, 0)),
            pl.BlockSpec((1, 1, _H), lambda e, t: (e, 0, 0)),
            pl.BlockSpec((1, 1, _H), lambda e, t: (e, 0, 0)),
            pl.BlockSpec((1, 1, 1), lambda e, t: (e, 0, 0)),
        ],
        out_specs=pl.BlockSpec((_N, 1), lambda e, t: (0, 0)),
        out_shape=jax.ShapeDtypeStruct((_N, 1), jnp.float32),
        scratch_shapes=[pltpu.VMEM((_N, 1), jnp.float32)],
    )(x, w_gate, W1, b1r, W2, b2r, w3r, b3r)
    return out


# speedup vs baseline: 2.1164x; 1.0983x over previous
import functools

import jax
import jax.numpy as jnp
from jax import lax
from jax.experimental import pallas as pl
from jax.experimental.pallas import tpu as pltpu
from jax.experimental.pallas import tpu_sc as plsc

_E = 8
_K = 2
_N = 2048
_D = 1024
_H = 1024

_BT = 256
_NB = _N * _K // _BT + _E
_NSLOT = _NB * _BT

_NC = 2
_NS = 16
_NW = _NC * _NS
_TPW = _N // _NW


def _router_body(x_ref, wg_ref, gates_ref, dest_ref, eid_ref, used_ref):
    x = x_ref[...]
    logits = jnp.dot(x, wg_ref[...], preferred_element_type=jnp.float32)
    m = jnp.max(logits, axis=1, keepdims=True)
    p = jnp.exp(logits - m)
    p = p / jnp.sum(p, axis=1, keepdims=True)

    lane = lax.broadcasted_iota(jnp.int32, p.shape, 1)
    v0 = jnp.max(p, axis=1, keepdims=True)
    e0 = jnp.min(jnp.where(p == v0, lane, _E), axis=1, keepdims=True)
    mask0 = lane == e0
    p1 = jnp.where(mask0, -1.0, p)
    v1 = jnp.max(p1, axis=1, keepdims=True)
    e1 = jnp.min(jnp.where(p1 == v1, lane, _E), axis=1, keepdims=True)
    mask1 = lane == e1

    denom = v0 + v1 + 1e-6
    gates_ref[:, 0:1] = v0 / denom
    gates_ref[:, 1:2] = v1 / denom

    ind = mask0.astype(jnp.int32) + mask1.astype(jnp.int32)
    s = ind
    sh = 1
    while sh < _N:
        top = jnp.zeros((sh, _E), jnp.int32)
        s = s + jnp.concatenate([top, s[: _N - sh, :]], axis=0)
        sh *= 2
    excl = s - ind
    cnt = s[_N - 1:_N, :]

    nb = (cnt + (_BT - 1)) // _BT
    s2 = nb
    sh = 1
    while sh < _E:
        left = jnp.zeros((1, sh), jnp.int32)
        s2 = s2 + jnp.concatenate([left, s2[:, : _E - sh]], axis=1)
        sh *= 2
    blkoff = s2 - nb

    base = blkoff * _BT
    base_b = jnp.broadcast_to(base, (_N, _E))
    rank0 = jnp.sum(jnp.where(mask0, excl + base_b, 0), axis=1, keepdims=True)
    rank1 = jnp.sum(jnp.where(mask1, excl + base_b, 0), axis=1, keepdims=True)
    dest_ref[:, 0:1] = rank0
    dest_ref[:, 1:2] = rank1

    blk = lax.broadcasted_iota(jnp.int32, (_NB, _E), 0)
    eid_ref[...] = jnp.sum((jnp.broadcast_to(blkoff, (_NB, _E)) <= blk)
                           .astype(jnp.int32), axis=1, keepdims=True) - 1
    used_ref[...] = blkoff[:, _E - 1:_E] + nb[:, _E - 1:_E]


def _router(x, w_gate):
    return pl.pallas_call(
        _router_body,
        in_specs=[
            pl.BlockSpec((_N, _D), lambda: (0, 0)),
            pl.BlockSpec((_D, _E), lambda: (0, 0)),
        ],
        out_specs=[
            pl.BlockSpec((_N, _K), lambda: (0, 0)),
            pl.BlockSpec((_N, _K), lambda: (0, 0)),
            pl.BlockSpec((_NB, 1), lambda: (0, 0)),
            pl.BlockSpec((1, 1), lambda: (0, 0)),
        ],
        out_shape=[
            jax.ShapeDtypeStruct((_N, _K), jnp.float32),
            jax.ShapeDtypeStruct((_N, _K), jnp.int32),
            jax.ShapeDtypeStruct((_NB, 1), jnp.int32),
            jax.ShapeDtypeStruct((1, 1), jnp.int32),
        ],
    )(x, w_gate)


def _dispatch(x, d0, d1):
    mesh = plsc.VectorSubcoreMesh(core_axis_name="c", subcore_axis_name="s")

    @functools.partial(
        pl.kernel, mesh=mesh,
        out_type=jax.ShapeDtypeStruct((_NSLOT, _D), jnp.float32),
        scratch_types=[
            pltpu.VMEM((_TPW,), jnp.int32),
            pltpu.VMEM((_TPW,), jnp.int32),
            pltpu.VMEM((_TPW, _D), jnp.float32),
            pltpu.SemaphoreType.DMA,
        ],
    )
    def k(x_hbm, d0_hbm, d1_hbm, xg_hbm, i0_v, i1_v, rows_v, sem):
        wid = lax.axis_index("s") * _NC + lax.axis_index("c")
        base = wid * _TPW
        pltpu.sync_copy(d0_hbm.at[pl.ds(base, _TPW)], i0_v)
        pltpu.sync_copy(d1_hbm.at[pl.ds(base, _TPW)], i1_v)
        pltpu.sync_copy(x_hbm.at[pl.ds(base, _TPW)], rows_v)
        c0 = pltpu.async_copy(rows_v, xg_hbm.at[i0_v], sem)
        c1 = pltpu.async_copy(rows_v, xg_hbm.at[i1_v], sem)
        c0.wait()
        c1.wait()

    return k(x, d0, d1)


def _experts_body(eid_ref, used_ref, xg_ref, w1_ref, b1_ref, w2_ref, b2_ref,
                  w3_ref, b3_ref, y_ref):
    i = pl.program_id(0)

    @pl.when(i < used_ref[0])
    def _compute():
        e = eid_ref[i]
        scale = lax.shift_left(1, e).astype(jnp.float32)
        h = jnp.tanh(jnp.dot(xg_ref[...] * scale, w1_ref[0],
                             preferred_element_type=jnp.float32) + b1_ref[0])
        h = jnp.tanh(jnp.dot(h, w2_ref[0],
                             preferred_element_type=jnp.float32) + b2_ref[0])
        y_ref[...] = (jnp.sum(h * w3_ref[0], axis=1, keepdims=True)
                      + b3_ref[0, 0, 0])


def _experts(xg, blk_eid, used, W1, b1r, W2, b2r, w3r, b3r):
    grid_spec = pltpu.PrefetchScalarGridSpec(
        num_scalar_prefetch=2,
        grid=(_NB,),
        in_specs=[
            pl.BlockSpec((_BT, _D), lambda i, eid, used: (i, 0)),
            pl.BlockSpec((1, _D, _H), lambda i, eid, used: (eid[i], 0, 0)),
            pl.BlockSpec((1, 1, _H), lambda i, eid, used: (eid[i], 0, 0)),
            pl.BlockSpec((1, _H, _H), lambda i, eid, used: (eid[i], 0, 0)),
            pl.BlockSpec((1, 1, _H), lambda i, eid, used: (eid[i], 0, 0)),
            pl.BlockSpec((1, 1, _H), lambda i, eid, used: (eid[i], 0, 0)),
            pl.BlockSpec((1, 1, 1), lambda i, eid, used: (eid[i], 0, 0)),
        ],
        out_specs=pl.BlockSpec((_BT, 1), lambda i, eid, used: (i, 0)),
    )
    return pl.pallas_call(
        _experts_body,
        grid_spec=grid_spec,
        out_shape=jax.ShapeDtypeStruct((_NSLOT, 1), jnp.float32),
    )(blk_eid, used, xg, W1, b1r, W2, b2r, w3r, b3r)


def _combine(y, d0, d1, g0, g1):
    mesh = plsc.VectorSubcoreMesh(core_axis_name="c", subcore_axis_name="s")

    @functools.partial(
        pl.kernel, mesh=mesh,
        out_type=jax.ShapeDtypeStruct((_N,), jnp.float32),
        scratch_types=[
            pltpu.VMEM((_TPW,), jnp.int32),
            pltpu.VMEM((_TPW,), jnp.int32),
            pltpu.VMEM((_TPW,), jnp.float32),
            pltpu.VMEM((_TPW,), jnp.float32),
            pltpu.VMEM((_TPW,), jnp.float32),
            pltpu.VMEM((_TPW,), jnp.float32),
            pltpu.VMEM((_TPW,), jnp.float32),
            pltpu.SemaphoreType.DMA,
        ],
    )
    def k(y_hbm, d0_hbm, d1_hbm, g0_hbm, g1_hbm, out_hbm,
          i0_v, i1_v, g0_v, g1_v, y0_v, y1_v, o_v, sem):
        wid = lax.axis_index("s") * _NC + lax.axis_index("c")
        base = wid * _TPW
        pltpu.sync_copy(d0_hbm.at[pl.ds(base, _TPW)], i0_v)
        pltpu.sync_copy(d1_hbm.at[pl.ds(base, _TPW)], i1_v)
        pltpu.sync_copy(g0_hbm.at[pl.ds(base, _TPW)], g0_v)
        pltpu.sync_copy(g1_hbm.at[pl.ds(base, _TPW)], g1_v)
        c0 = pltpu.async_copy(y_hbm.at[i0_v], y0_v, sem)
        c1 = pltpu.async_copy(y_hbm.at[i1_v], y1_v, sem)
        c0.wait()
        c1.wait()
        for j in range(_TPW // 16):
            sl = pl.ds(j * 16, 16)
            o_v[sl] = g0_v[sl] * y0_v[sl] + g1_v[sl] * y1_v[sl]
        pltpu.sync_copy(o_v, out_hbm.at[pl.ds(base, _TPW)])

    return k(y, d0, d1, g0, g1)


@jax.jit
def kernel(x, w_gate, W1, b1, W2, b2, W3, b3):
    w3r = W3[:, :, 0].reshape(_E, 1, _H)
    b1r = b1.reshape(_E, 1, _H)
    b2r = b2.reshape(_E, 1, _H)
    b3r = b3.reshape(_E, 1, 1)

    gates, dest, blk_eid, used = _router(x, w_gate)
    d0 = dest[:, 0]
    d1 = dest[:, 1]
    xg = _dispatch(x, d0, d1)
    y = _experts(xg, blk_eid.reshape(_NB), used.reshape(1), W1, b1r, W2, b2r,
                 w3r, b3r)
    out = _combine(y.reshape(_NSLOT), d0, d1, gates[:, 0], gates[:, 1])
    return out.reshape(_N, 1)
